# Initial kernel scaffold; baseline (speedup 1.0000x reference)
import jax
import jax.numpy as jnp
from jax.experimental import pallas as pl


def kernel(x, output_idx, process_indices, process_neurons, output_neurons):
    raise NotImplementedError("write your pallas kernel here")


# speedup vs baseline: 12.0831x; 12.0831x over previous
import jax
import jax.numpy as jnp
from jax.experimental import pallas as pl
from jax.experimental.pallas import tpu as pltpu

_EPS = 1e-08


def _tc_body(xs_ref, oi_ref, pi0_ref, pi1_ref, pn_ref, w_ref, out_ref):
    xs = xs_ref[...]
    pn = pn_ref[...]
    pi0 = pi0_ref[...]
    pi1 = pi1_ref[...]
    oi = oi_ref[...]

    s, r = xs.shape
    np_, _ = pn.shape

    iota_np = jax.lax.broadcasted_iota(jnp.int32, (s, np_), 1)
    oh0 = (pi0 == iota_np).astype(jnp.float32)
    oh1 = (pi1 == iota_np).astype(jnp.float32)
    v0 = jnp.dot(oh0, pn, preferred_element_type=jnp.float32)
    v1 = jnp.dot(oh1, pn, preferred_element_type=jnp.float32)

    s0 = jnp.sum(v0 * v0, axis=1, keepdims=True) + _EPS
    s1 = jnp.sum(v1 * v1, axis=1, keepdims=True) + _EPS
    a = jnp.sum(v0 * xs, axis=1, keepdims=True)
    b = jnp.sum(v1 * xs, axis=1, keepdims=True)
    d = jnp.sum(v0 * v1, axis=1, keepdims=True)
    c0 = 2.0 * a / s0
    c1 = 2.0 * (b - c0 * d) / s1
    x2 = xs - c0 * v0 - c1 * v1

    n_out = w_ref.shape[0] // r
    iota_e = jax.lax.broadcasted_iota(jnp.int32, (s, n_out), 1)
    ohe = (oi == iota_e).astype(jnp.float32)
    xb = jnp.concatenate(
        [x2 * ohe[:, e:e + 1] for e in range(n_out)], axis=1)
    out_ref[...] = jnp.dot(xb, w_ref[...], preferred_element_type=jnp.float32)


def kernel(x, output_idx, process_indices, process_neurons, output_neurons):
    b, s, r = x.shape
    n_out, _, d_model = output_neurons.shape
    xs = x.reshape(b * s, r)
    oi = output_idx.reshape(b * s, 1).astype(jnp.int32)
    pi0 = process_indices[..., 0].reshape(b * s, 1).astype(jnp.int32)
    pi1 = process_indices[..., 1].reshape(b * s, 1).astype(jnp.int32)
    w = output_neurons.reshape(n_out * r, d_model)

    out = pl.pallas_call(
        _tc_body,
        out_shape=jax.ShapeDtypeStruct((b * s, d_model), jnp.float32),
    )(xs, oi, pi0, pi1, process_neurons, w)
    return out.reshape(b, s, d_model)
